# baseline (speedup 1.0000x reference)
import functools

import jax
import jax.numpy as jnp
from jax import lax
from jax.experimental import pallas as pl
from jax.experimental.pallas import tpu as pltpu
from jax.experimental.pallas import tpu_sc as plsc

N = 10000
D = 128
HD = D // 2
MSG = 32
NDEG = 11
E = 320000

NC, NS, K = 2, 16, 128
NW = NC * NS
NH = 2
CHH = 80
CH = NH * CHH
EPT = CH * K
NB = 4
NPAD = 10112
RPT = NPAD // NS

_sc_mesh = plsc.VectorSubcoreMesh(core_axis_name="c", subcore_axis_name="s")


@functools.partial(
    pl.kernel,
    out_type=(
        jax.ShapeDtypeStruct((NC * NPAD, HD), jnp.float32),
        jax.ShapeDtypeStruct((NC * NPAD, 8), jnp.float32),
    ),
    mesh=_sc_mesh,
    compiler_params=pltpu.CompilerParams(use_tc_tiling_on_sc=False),
    scratch_types=[
        pltpu.VMEM((CHH, K), jnp.int32),
        pltpu.VMEM((CHH, K), jnp.int32),
        [pltpu.VMEM((K, HD), jnp.float32) for _ in range(NB)],
        pltpu.VMEM((K, 8), jnp.float32),
        pltpu.VMEM_SHARED((NPAD, HD), jnp.float32),
        pltpu.VMEM_SHARED((NPAD, 8), jnp.float32),
        [pltpu.SemaphoreType.DMA for _ in range(NB)],
        [pltpu.SemaphoreType.DMA for _ in range(NB)],
    ],
)
def _sc_segment_sum(x_hbm, src_hbm, dst_hbm, zh_hbm, zd_hbm, oh_hbm,
                    ph_hbm, pd_hbm,
                    src_v, dst_v, bufs, ones_v, ha, dacc, gsems, ssems):
    c = lax.axis_index("c")
    s = lax.axis_index("s")
    wid = c * NS + s
    r0 = s * RPT

    pltpu.sync_copy(zh_hbm, ha.at[pl.ds(r0, RPT)])
    pltpu.sync_copy(zd_hbm, dacc.at[pl.ds(r0, RPT)])
    pltpu.sync_copy(oh_hbm, ones_v)
    plsc.subcore_barrier()

    def wait_scatter(j, b, par):
        pltpu.make_async_copy(bufs[b], ha.at[dst_v.at[j]], ssems[b]).wait()

        @pl.when(c == par)
        def _():
            pltpu.make_async_copy(ones_v, dacc.at[dst_v.at[j]], ssems[b]).wait()

    for half in range(NH):
        pltpu.sync_copy(src_hbm.at[wid * NH + half], src_v)
        pltpu.sync_copy(dst_hbm.at[wid * NH + half], dst_v)
        pltpu.async_copy(x_hbm.at[src_v.at[0]], bufs[0], gsems[0])
        pltpu.async_copy(x_hbm.at[src_v.at[1]], bufs[1], gsems[1])

        def quad(t, carry):
            for b in range(NB):
                j = t * NB + b
                pltpu.make_async_copy(x_hbm.at[src_v.at[j]], bufs[b],
                                      gsems[b]).wait()
                pltpu.async_copy(bufs[b], ha.at[dst_v.at[j]], ssems[b],
                                 add=True)

                @pl.when(c == b % 2)
                def _():
                    pltpu.async_copy(ones_v, dacc.at[dst_v.at[j]], ssems[b],
                                     add=True)

                @pl.when(j >= 2)
                def _():
                    wait_scatter(j - 2, (b - 2) % NB, b % 2)

                bn = (b + 2) % NB

                @pl.when(j + 2 < CHH)
                def _():
                    pltpu.async_copy(x_hbm.at[src_v.at[j + 2]], bufs[bn],
                                     gsems[bn])

            return carry

        lax.fori_loop(0, CHH // NB, quad, 0)
        wait_scatter(CHH - 2, (CHH - 2) % NB, (CHH - 2) % 2)
        wait_scatter(CHH - 1, (CHH - 1) % NB, (CHH - 1) % 2)

    plsc.subcore_barrier()
    pltpu.sync_copy(ha.at[pl.ds(r0, RPT)], ph_hbm.at[pl.ds(c * NPAD + r0, RPT)])
    pltpu.sync_copy(dacc.at[pl.ds(r0, RPT)],
                    pd_hbm.at[pl.ds(c * NPAD + r0, RPT)])


GB = 8
BR = NPAD // GB


def _tc_body(ph_ref, pd_ref, x_ref, ah_ref, b_ref, bc_ref, w1_ref, b1_ref,
             w2_ref, b2_ref, emb_ref, out_ref):
    xb = x_ref[...]
    r = (jnp.dot(ph_ref[0], ah_ref[0], preferred_element_type=jnp.float32)
         + jnp.dot(ph_ref[1], ah_ref[1], preferred_element_type=jnp.float32)
         + jnp.dot(xb, b_ref[...], preferred_element_type=jnp.float32)
         + bc_ref[...])
    dsum = pd_ref[0] + pd_ref[1]
    deg = jnp.minimum(dsum[:, 0:1], float(NDEG - 1))
    conv = jnp.zeros((BR, MSG), jnp.float32)
    for i in range(NDEG):
        conv = jnp.where(deg == float(i), r[:, i * MSG:(i + 1) * MSG], conv)
    emb_ref[...] = conv
    t = jnp.maximum(conv, 0.0)
    t = jnp.dot(t, w1_ref[...], preferred_element_type=jnp.float32) + b1_ref[...]
    out_ref[...] = (jnp.dot(t, w2_ref[...], preferred_element_type=jnp.float32)
                    + b2_ref[...])


_tc_dense = pl.pallas_call(
    _tc_body,
    grid=(GB,),
    in_specs=[
        pl.BlockSpec((NC, BR, HD), lambda g: (0, g, 0)),
        pl.BlockSpec((NC, BR, 8), lambda g: (0, g, 0)),
        pl.BlockSpec((BR, D), lambda g: (g, 0)),
        pl.BlockSpec((NC, HD, NDEG * MSG), lambda g: (0, 0, 0)),
        pl.BlockSpec((D, NDEG * MSG), lambda g: (0, 0)),
        pl.BlockSpec((NDEG * MSG,), lambda g: (0,)),
        pl.BlockSpec((MSG, MSG), lambda g: (0, 0)),
        pl.BlockSpec((MSG,), lambda g: (0,)),
        pl.BlockSpec((MSG, MSG), lambda g: (0, 0)),
        pl.BlockSpec((MSG,), lambda g: (0,)),
    ],
    out_specs=[
        pl.BlockSpec((BR, MSG), lambda g: (g, 0)),
        pl.BlockSpec((BR, MSG), lambda g: (g, 0)),
    ],
    out_shape=[
        jax.ShapeDtypeStruct((NPAD, MSG), jnp.float32),
        jax.ShapeDtypeStruct((NPAD, MSG), jnp.float32),
    ],
)


def kernel(x, edge_index, batch, Wl, bl, Wr, W1, b1, W2, b2):
    src = edge_index[0]
    dst = edge_index[1]
    pad = NS * EPT - E
    srcp = jnp.concatenate([src, jnp.zeros((pad,), jnp.int32)])
    dstp = jnp.concatenate([dst, jnp.full((pad,), N, jnp.int32)])
    src2 = srcp.reshape(NS * NH, CHH, K)
    dst2 = dstp.reshape(NS * NH, CHH, K)
    src4 = jnp.concatenate([src2, src2 + N], axis=0)
    dst4 = jnp.concatenate([dst2, dst2], axis=0)
    xcat = jnp.concatenate([x[:, :HD], x[:, HD:]], axis=0)
    zh = jnp.zeros((RPT, HD), jnp.float32)
    zd = jnp.zeros((RPT, 8), jnp.float32)
    oh = jnp.ones((K, 8), jnp.float32)

    ph, pd = _sc_segment_sum(xcat, src4, dst4, zh, zd, oh)

    x_pad = jnp.concatenate([x, jnp.zeros((NPAD - N, D), jnp.float32)])
    a = jnp.transpose(Wl, (2, 0, 1)).reshape(D, NDEG * MSG)
    ah = jnp.stack([a[:HD], a[HD:]])
    b = jnp.transpose(Wr, (2, 0, 1)).reshape(D, NDEG * MSG)
    bc = bl.reshape(NDEG * MSG)

    emb, out = _tc_dense(ph.reshape(NC, NPAD, HD), pd.reshape(NC, NPAD, 8),
                         x_pad, ah, b, bc, W1.T, b1, W2.T, b2)
    return emb[:N], out[:N]


# speedup vs baseline: 1.7449x; 1.7449x over previous
import functools

import jax
import jax.numpy as jnp
from jax import lax
from jax.experimental import pallas as pl
from jax.experimental.pallas import tpu as pltpu
from jax.experimental.pallas import tpu_sc as plsc

N = 10000
D = 128
XW = 136
MSG = 32
NDEG = 11
E = 320000

NC, NS, K = 2, 16, 80
NW = NC * NS
NSC = 5
SCH = 25
CH = NSC * SCH
EPT = CH * K
NPAD = 10112
RPT = NPAD // NS

_sc_mesh = plsc.VectorSubcoreMesh(core_axis_name="c", subcore_axis_name="s")


@functools.partial(
    pl.kernel,
    out_type=jax.ShapeDtypeStruct((NC * NPAD, XW), jnp.float32),
    mesh=_sc_mesh,
    compiler_params=pltpu.CompilerParams(use_tc_tiling_on_sc=False),
    scratch_types=[
        pltpu.VMEM((SCH, K), jnp.int32),
        pltpu.VMEM((SCH, K), jnp.int32),
        pltpu.VMEM((K, XW), jnp.float32),
        pltpu.VMEM((K, XW), jnp.float32),
        pltpu.VMEM_SHARED((NPAD, XW), jnp.float32),
        pltpu.SemaphoreType.DMA,
        pltpu.SemaphoreType.DMA,
    ],
)
def _sc_segment_sum(x_hbm, src_hbm, dst_hbm, zh_hbm,
                    ph_hbm,
                    src_v, dst_v, buf0, buf1, ha, sem0, sem1):
    c = lax.axis_index("c")
    s = lax.axis_index("s")
    wid = c * NS + s
    r0 = s * RPT

    pltpu.sync_copy(zh_hbm, ha.at[pl.ds(r0, RPT)])
    plsc.subcore_barrier()

    for g in range(NSC):
        pltpu.sync_copy(src_hbm.at[wid * NSC + g], src_v)
        pltpu.sync_copy(dst_hbm.at[wid * NSC + g], dst_v)
        pltpu.async_copy(x_hbm.at[src_v.at[0]], buf0, sem0)

        def step(t, carry):
            j0 = t * 2
            j1 = j0 + 1
            pltpu.async_copy(x_hbm.at[src_v.at[j1]], buf1, sem1)
            pltpu.make_async_copy(x_hbm.at[src_v.at[j0]], buf0, sem0).wait()
            pltpu.sync_copy(buf0, ha.at[dst_v.at[j0]], add=True)
            pltpu.async_copy(x_hbm.at[src_v.at[j0 + 2]], buf0, sem0)
            pltpu.make_async_copy(x_hbm.at[src_v.at[j1]], buf1, sem1).wait()
            pltpu.sync_copy(buf1, ha.at[dst_v.at[j1]], add=True)
            return carry

        lax.fori_loop(0, SCH // 2, step, 0)
        pltpu.make_async_copy(x_hbm.at[src_v.at[SCH - 1]], buf0, sem0).wait()
        pltpu.sync_copy(buf0, ha.at[dst_v.at[SCH - 1]], add=True)

    plsc.subcore_barrier()
    pltpu.sync_copy(ha.at[pl.ds(r0, RPT)], ph_hbm.at[pl.ds(c * NPAD + r0, RPT)])


GB = 8
BR = NPAD // GB


def _tc_body(ph_ref, x_ref, a_ref, b_ref, bc_ref, w1_ref, b1_ref,
             w2_ref, b2_ref, emb_ref, out_ref):
    hw = ph_ref[0] + ph_ref[1]
    h = hw[:, :D]
    deg = jnp.minimum(hw[:, D:D + 1], float(NDEG - 1))
    xb = x_ref[...]
    r = (jnp.dot(h, a_ref[...], preferred_element_type=jnp.float32)
         + jnp.dot(xb, b_ref[...], preferred_element_type=jnp.float32)
         + bc_ref[...])
    conv = jnp.zeros((BR, MSG), jnp.float32)
    for i in range(NDEG):
        conv = jnp.where(deg == float(i), r[:, i * MSG:(i + 1) * MSG], conv)
    emb_ref[...] = conv
    t = jnp.maximum(conv, 0.0)
    t = jnp.dot(t, w1_ref[...], preferred_element_type=jnp.float32) + b1_ref[...]
    out_ref[...] = (jnp.dot(t, w2_ref[...], preferred_element_type=jnp.float32)
                    + b2_ref[...])


_tc_dense = pl.pallas_call(
    _tc_body,
    grid=(GB,),
    in_specs=[
        pl.BlockSpec((NC, BR, XW), lambda g: (0, g, 0)),
        pl.BlockSpec((BR, D), lambda g: (g, 0)),
        pl.BlockSpec((D, NDEG * MSG), lambda g: (0, 0)),
        pl.BlockSpec((D, NDEG * MSG), lambda g: (0, 0)),
        pl.BlockSpec((NDEG * MSG,), lambda g: (0,)),
        pl.BlockSpec((MSG, MSG), lambda g: (0, 0)),
        pl.BlockSpec((MSG,), lambda g: (0,)),
        pl.BlockSpec((MSG, MSG), lambda g: (0, 0)),
        pl.BlockSpec((MSG,), lambda g: (0,)),
    ],
    out_specs=[
        pl.BlockSpec((BR, MSG), lambda g: (g, 0)),
        pl.BlockSpec((BR, MSG), lambda g: (g, 0)),
    ],
    out_shape=[
        jax.ShapeDtypeStruct((NPAD, MSG), jnp.float32),
        jax.ShapeDtypeStruct((NPAD, MSG), jnp.float32),
    ],
)


def kernel(x, edge_index, batch, Wl, bl, Wr, W1, b1, W2, b2):
    src = edge_index[0]
    dst = edge_index[1]
    src5 = src.reshape(NW * NSC, SCH, K)
    dst5 = dst.reshape(NW * NSC, SCH, K)
    ones_col = jnp.ones((N, 1), jnp.float32)
    pad_cols = jnp.zeros((N, XW - D - 1), jnp.float32)
    xa = jnp.concatenate([x, ones_col, pad_cols], axis=1)
    zh = jnp.zeros((RPT, XW), jnp.float32)

    ph = _sc_segment_sum(xa, src5, dst5, zh)

    x_pad = jnp.concatenate([x, jnp.zeros((NPAD - N, D), jnp.float32)])
    a = jnp.transpose(Wl, (2, 0, 1)).reshape(D, NDEG * MSG)
    b = jnp.transpose(Wr, (2, 0, 1)).reshape(D, NDEG * MSG)
    bc = bl.reshape(NDEG * MSG)

    emb, out = _tc_dense(ph.reshape(NC, NPAD, XW), x_pad, a, b, bc,
                         W1.T, b1, W2.T, b2)
    return emb[:N], out[:N]


# speedup vs baseline: 1.7760x; 1.0178x over previous
import functools

import jax
import jax.numpy as jnp
from jax import lax
from jax.experimental import pallas as pl
from jax.experimental.pallas import tpu as pltpu
from jax.experimental.pallas import tpu_sc as plsc

N = 10000
D = 128
XW = 136
MSG = 32
NDEG = 11
E = 320000

NC, NS, K = 2, 16, 80
NW = NC * NS
NSC = 5
SCH = 25
CH = NSC * SCH
EPT = CH * K
NPAD = 10112
RPT = NPAD // NS

_sc_mesh = plsc.VectorSubcoreMesh(core_axis_name="c", subcore_axis_name="s")


@functools.partial(
    pl.kernel,
    out_type=jax.ShapeDtypeStruct((NC * NPAD, XW), jnp.float32),
    mesh=_sc_mesh,
    compiler_params=pltpu.CompilerParams(use_tc_tiling_on_sc=False),
    scratch_types=[
        pltpu.VMEM((SCH, K), jnp.int32),
        pltpu.VMEM((SCH, K), jnp.int32),
        pltpu.VMEM((K, XW), jnp.float32),
        pltpu.VMEM((K, XW), jnp.float32),
        pltpu.VMEM_SHARED((NPAD, XW), jnp.float32),
        pltpu.SemaphoreType.DMA,
        pltpu.SemaphoreType.DMA,
    ],
)
def _sc_segment_sum(x_hbm, src_hbm, dst_hbm, zh_hbm,
                    ph_hbm,
                    src_v, dst_v, buf0, buf1, ha, sem0, sem1):
    c = lax.axis_index("c")
    s = lax.axis_index("s")
    wid = c * NS + s
    r0 = s * RPT

    pltpu.sync_copy(zh_hbm, ha.at[pl.ds(r0, RPT)])
    plsc.subcore_barrier()

    for g in range(NSC):
        pltpu.sync_copy(src_hbm.at[wid * NSC + g], src_v)
        pltpu.sync_copy(dst_hbm.at[wid * NSC + g], dst_v)
        pltpu.async_copy(x_hbm.at[src_v.at[0]], buf0, sem0)

        def step(t, carry):
            j0 = t * 2
            j1 = j0 + 1
            pltpu.async_copy(x_hbm.at[src_v.at[j1]], buf1, sem1)
            pltpu.make_async_copy(x_hbm.at[src_v.at[j0]], buf0, sem0).wait()
            pltpu.sync_copy(buf0, ha.at[dst_v.at[j0]], add=True)
            pltpu.async_copy(x_hbm.at[src_v.at[j0 + 2]], buf0, sem0)
            pltpu.make_async_copy(x_hbm.at[src_v.at[j1]], buf1, sem1).wait()
            pltpu.sync_copy(buf1, ha.at[dst_v.at[j1]], add=True)
            return carry

        lax.fori_loop(0, SCH // 2, step, 0)
        pltpu.make_async_copy(x_hbm.at[src_v.at[SCH - 1]], buf0, sem0).wait()
        pltpu.sync_copy(buf0, ha.at[dst_v.at[SCH - 1]], add=True)

    plsc.subcore_barrier()
    pltpu.sync_copy(ha.at[pl.ds(r0, RPT)], ph_hbm.at[pl.ds(c * NPAD + r0, RPT)])


GB = 10
BR = N // GB


def _tc_body(ph_ref, x_ref, a_ref, b_ref, bc_ref, w1_ref, b1_ref,
             w2_ref, b2_ref, emb_ref, out_ref):
    hw = ph_ref[0] + ph_ref[1]
    h = hw[:, :D]
    deg = jnp.minimum(hw[:, D:D + 1], float(NDEG - 1))
    xb = x_ref[...]
    r = (jnp.dot(h, a_ref[...], preferred_element_type=jnp.float32)
         + jnp.dot(xb, b_ref[...], preferred_element_type=jnp.float32)
         + bc_ref[...])
    conv = jnp.zeros((BR, MSG), jnp.float32)
    for i in range(NDEG):
        conv = jnp.where(deg == float(i), r[:, i * MSG:(i + 1) * MSG], conv)
    emb_ref[...] = conv
    t = jnp.maximum(conv, 0.0)
    t = jnp.dot(t, w1_ref[...], preferred_element_type=jnp.float32) + b1_ref[...]
    out_ref[...] = (jnp.dot(t, w2_ref[...], preferred_element_type=jnp.float32)
                    + b2_ref[...])


_tc_dense = pl.pallas_call(
    _tc_body,
    grid=(GB,),
    in_specs=[
        pl.BlockSpec((NC, BR, XW), lambda g: (0, g, 0)),
        pl.BlockSpec((BR, D), lambda g: (g, 0)),
        pl.BlockSpec((D, NDEG * MSG), lambda g: (0, 0)),
        pl.BlockSpec((D, NDEG * MSG), lambda g: (0, 0)),
        pl.BlockSpec((NDEG * MSG,), lambda g: (0,)),
        pl.BlockSpec((MSG, MSG), lambda g: (0, 0)),
        pl.BlockSpec((MSG,), lambda g: (0,)),
        pl.BlockSpec((MSG, MSG), lambda g: (0, 0)),
        pl.BlockSpec((MSG,), lambda g: (0,)),
    ],
    out_specs=[
        pl.BlockSpec((BR, MSG), lambda g: (g, 0)),
        pl.BlockSpec((BR, MSG), lambda g: (g, 0)),
    ],
    out_shape=[
        jax.ShapeDtypeStruct((N, MSG), jnp.float32),
        jax.ShapeDtypeStruct((N, MSG), jnp.float32),
    ],
)


def kernel(x, edge_index, batch, Wl, bl, Wr, W1, b1, W2, b2):
    src = edge_index[0]
    dst = edge_index[1]
    src5 = src.reshape(NW * NSC, SCH, K)
    dst5 = dst.reshape(NW * NSC, SCH, K)
    ones_col = jnp.ones((N, 1), jnp.float32)
    pad_cols = jnp.zeros((N, XW - D - 1), jnp.float32)
    xa = jnp.concatenate([x, ones_col, pad_cols], axis=1)
    zh = jnp.zeros((RPT, XW), jnp.float32)

    ph = _sc_segment_sum(xa, src5, dst5, zh)

    a = jnp.transpose(Wl, (2, 0, 1)).reshape(D, NDEG * MSG)
    b = jnp.transpose(Wr, (2, 0, 1)).reshape(D, NDEG * MSG)
    bc = bl.reshape(NDEG * MSG)

    emb, out = _tc_dense(ph.reshape(NC, NPAD, XW), x, a, b, bc,
                         W1.T, b1, W2.T, b2)
    return emb, out
